# baseline (speedup 1.0000x reference)
import jax
import jax.numpy as jnp
from jax.experimental import pallas as pl
from jax.experimental.pallas import tpu as pltpu

_BLOCK_ROWS = 4096


def _copy_block(emb_ref, out_ref):
    out_ref[...] = emb_ref[...]


def kernel(inputs, embeddings):
    seq_len = inputs.shape[1]
    emb_dim = embeddings.shape[1]
    table = embeddings[:seq_len, :]
    blk = min(_BLOCK_ROWS, seq_len)
    grid = (seq_len // blk,)
    out = pl.pallas_call(
        _copy_block,
        grid=grid,
        in_specs=[pl.BlockSpec((blk, emb_dim), lambda i: (i, 0))],
        out_specs=pl.BlockSpec((blk, emb_dim), lambda i: (i, 0)),
        out_shape=jax.ShapeDtypeStruct((seq_len, emb_dim), embeddings.dtype),
        compiler_params=pltpu.CompilerParams(
            dimension_semantics=("parallel",),
        ),
    )(table)
    return out[None]
